# baseline (speedup 1.0000x reference)
import functools
import math

import jax
import jax.numpy as jnp
from jax import lax
from jax.experimental import pallas as pl
from jax.experimental.pallas import tpu as pltpu
from jax.experimental.pallas import tpu_sc as plsc

D_MODEL = 64
SCALE = math.sqrt(D_MODEL)

NUM_CORES = 2
NUM_SUBCORES = 16
NUM_WORKERS = NUM_CORES * NUM_SUBCORES
LANES = 16

CHUNK = 512
GATHER = 128


def _emb_body(n_chunks, b_per_w, x_hbm, table_hbm, out_hbm, idx_v, rows_v, sem):
    wid = lax.axis_index("s") * NUM_CORES + lax.axis_index("c")
    base = wid * b_per_w

    def chunk_body(g, carry):
        row0 = base + g * CHUNK
        pltpu.sync_copy(x_hbm.at[pl.ds(row0, CHUNK)], idx_v)
        copies = []
        for j in range(CHUNK // GATHER):
            copies.append(
                pltpu.async_copy(
                    table_hbm.at[idx_v.at[pl.ds(j * GATHER, GATHER)]],
                    rows_v.at[pl.ds(j * GATHER, GATHER)],
                    sem,
                )
            )
        for c in copies:
            c.wait()

        def row_body(r, rcarry):
            for c4 in range(D_MODEL // LANES):
                sl = pl.ds(c4 * LANES, LANES)
                rows_v[r, sl] = rows_v[r, sl] * SCALE
            return rcarry

        lax.fori_loop(0, CHUNK, row_body, 0)
        pltpu.sync_copy(rows_v, out_hbm.at[pl.ds(row0, CHUNK)])
        return carry

    lax.fori_loop(0, n_chunks, chunk_body, 0)


def kernel(x, table):
    n_rows, n_cols = x.shape
    B = n_rows * n_cols
    xf = x.reshape(B).astype(jnp.int32)

    b_per_w = B // NUM_WORKERS
    n_chunks = b_per_w // CHUNK
    assert b_per_w * NUM_WORKERS == B and n_chunks * CHUNK == b_per_w

    mesh = plsc.VectorSubcoreMesh(core_axis_name="c", subcore_axis_name="s")
    emb = pl.kernel(
        functools.partial(_emb_body, n_chunks, b_per_w),
        out_type=jax.ShapeDtypeStruct((B, D_MODEL), jnp.float32),
        mesh=mesh,
        scratch_types=[
            pltpu.VMEM((CHUNK,), jnp.int32),
            pltpu.VMEM((CHUNK, D_MODEL), jnp.float32),
            pltpu.SemaphoreType.DMA,
        ],
        compiler_params=pltpu.CompilerParams(use_tc_tiling_on_sc=False),
    )
    out = emb(xf, table)
    return out.reshape(n_rows, n_cols, D_MODEL)


# speedup vs baseline: 1.2185x; 1.2185x over previous
import functools
import math

import jax
import jax.numpy as jnp
from jax import lax
from jax.experimental import pallas as pl
from jax.experimental.pallas import tpu as pltpu
from jax.experimental.pallas import tpu_sc as plsc

D_MODEL = 64
SCALE = math.sqrt(D_MODEL)

NUM_CORES = 2
NUM_SUBCORES = 16
NUM_WORKERS = NUM_CORES * NUM_SUBCORES
LANES = 16

CHUNK = 256
GATHER = 128
NBUF = 4


def _emb_body(n_chunks, b_per_w, x_hbm, table_hbm, out_hbm, idx_v, rows_v,
              *sems):
    isems = sems[0:NBUF]
    gsems = sems[NBUF:2 * NBUF]
    osems = sems[2 * NBUF:3 * NBUF]

    wid = lax.axis_index("s") * NUM_CORES + lax.axis_index("c")
    base = wid * b_per_w
    n = n_chunks

    def fire_idx(g, b):
        pltpu.async_copy(x_hbm.at[pl.ds(base + g * CHUNK, CHUNK)],
                         idx_v.at[b], isems[b])

    def wait_idx(b):
        pltpu.make_async_copy(x_hbm.at[pl.ds(0, CHUNK)], idx_v.at[b],
                              isems[b]).wait()

    def fire_gathers(g, b):
        for j in range(CHUNK // GATHER):
            pltpu.async_copy(
                table_hbm.at[idx_v.at[b, pl.ds(j * GATHER, GATHER)]],
                rows_v.at[b, pl.ds(j * GATHER, GATHER)],
                gsems[b],
            )

    def wait_gathers(b):
        pltpu.make_async_copy(out_hbm.at[pl.ds(0, CHUNK)], rows_v.at[b],
                              gsems[b]).wait()

    def fire_writeout(g, b):
        pltpu.async_copy(rows_v.at[b],
                         out_hbm.at[pl.ds(base + g * CHUNK, CHUNK)], osems[b])

    def wait_writeout(b):
        pltpu.make_async_copy(rows_v.at[b], out_hbm.at[pl.ds(0, CHUNK)],
                              osems[b]).wait()

    fire_idx(0, 0)
    fire_idx(1, 1)
    wait_idx(0)
    fire_gathers(0, 0)

    def outer(i, carry):
        for b in range(NBUF):
            g = i * NBUF + b
            bp1 = (b + 1) % NBUF
            bp2 = (b + 2) % NBUF

            @pl.when(g + 2 < n)
            def _():
                fire_idx(g + 2, bp2)

            @pl.when(g + 1 < n)
            def _():
                wait_idx(bp1)

                @pl.when(g + 1 >= NBUF)
                def _():
                    wait_writeout(bp1)

                fire_gathers(g + 1, bp1)

            wait_gathers(b)

            @plsc.parallel_loop(0, CHUNK, 1, unroll=8)
            def _(r):
                for c4 in range(D_MODEL // LANES):
                    sl = pl.ds(c4 * LANES, LANES)
                    rows_v[b, r, sl] = rows_v[b, r, sl] * SCALE

            fire_writeout(g, b)
        return carry

    lax.fori_loop(0, n // NBUF, outer, 0)

    for b in range(NBUF):
        wait_writeout(b)


def kernel(x, table):
    n_rows, n_cols = x.shape
    B = n_rows * n_cols
    xf = x.reshape(B).astype(jnp.int32)

    b_per_w = B // NUM_WORKERS
    n_chunks = b_per_w // CHUNK
    assert b_per_w * NUM_WORKERS == B and n_chunks * CHUNK == b_per_w
    assert n_chunks % NBUF == 0

    mesh = plsc.VectorSubcoreMesh(core_axis_name="c", subcore_axis_name="s")
    emb = pl.kernel(
        functools.partial(_emb_body, n_chunks, b_per_w),
        out_type=jax.ShapeDtypeStruct((B, D_MODEL), jnp.float32),
        mesh=mesh,
        scratch_types=(
            [pltpu.VMEM((NBUF, CHUNK), jnp.int32),
             pltpu.VMEM((NBUF, CHUNK, D_MODEL), jnp.float32)]
            + [pltpu.SemaphoreType.DMA] * (3 * NBUF)
        ),
        compiler_params=pltpu.CompilerParams(use_tc_tiling_on_sc=False),
    )
    out = emb(xf, table)
    return out.reshape(n_rows, n_cols, D_MODEL)
